# baseline (speedup 1.0000x reference)
import jax
import jax.numpy as jnp
from jax.experimental import pallas as pl
from jax.experimental.pallas import tpu as pltpu

_BLOCK_ROWS = 512
_IN_BUFS = 4


def _copy_body(t_ref, o_ref):
    o_ref[...] = t_ref[...]


def kernel(x, table):
    seq_len = x.shape[1]
    dim = table.shape[1]
    nblocks = seq_len // _BLOCK_ROWS

    def outer(t_hbm, o_hbm):
        pltpu.emit_pipeline(
            _copy_body,
            grid=(nblocks,),
            in_specs=[
                pl.BlockSpec(
                    (_BLOCK_ROWS, dim),
                    lambda i: (i, 0),
                    pipeline_mode=pl.Buffered(buffer_count=_IN_BUFS),
                )
            ],
            out_specs=[
                pl.BlockSpec(
                    (_BLOCK_ROWS, dim),
                    lambda i: (i, 0),
                    pipeline_mode=pl.Buffered(buffer_count=2),
                )
            ],
        )(t_hbm, o_hbm)

    return pl.pallas_call(
        outer,
        in_specs=[pl.BlockSpec(memory_space=pl.ANY)],
        out_specs=pl.BlockSpec(memory_space=pl.ANY),
        out_shape=jax.ShapeDtypeStruct((seq_len, dim), table.dtype),
    )(table)


# speedup vs baseline: 1.0381x; 1.0381x over previous
import jax
import jax.numpy as jnp
from jax.experimental import pallas as pl
from jax.experimental.pallas import tpu as pltpu

_BLOCK_ROWS = 1024
_IN_BUFS = 3


def _copy_body(t_ref, o_ref):
    o_ref[...] = t_ref[...]


def kernel(x, table):
    seq_len = x.shape[1]
    dim = table.shape[1]
    nblocks = seq_len // _BLOCK_ROWS

    def outer(t_hbm, o_hbm):
        pltpu.emit_pipeline(
            _copy_body,
            grid=(nblocks,),
            in_specs=[
                pl.BlockSpec(
                    (_BLOCK_ROWS, dim),
                    lambda i: (i, 0),
                    pipeline_mode=pl.Buffered(buffer_count=_IN_BUFS),
                )
            ],
            out_specs=[
                pl.BlockSpec(
                    (_BLOCK_ROWS, dim),
                    lambda i: (i, 0),
                    pipeline_mode=pl.Buffered(buffer_count=2),
                )
            ],
        )(t_hbm, o_hbm)

    return pl.pallas_call(
        outer,
        in_specs=[pl.BlockSpec(memory_space=pl.ANY)],
        out_specs=pl.BlockSpec(memory_space=pl.ANY),
        out_shape=jax.ShapeDtypeStruct((seq_len, dim), table.dtype),
    )(table)
